# baseline (speedup 1.0000x reference)
import functools

import jax
import jax.numpy as jnp
from jax import lax
from jax.experimental import pallas as pl
from jax.experimental.pallas import tpu as pltpu
from jax.experimental.pallas import tpu_sc as plsc

NB_, NT = 4096, 200
D = 64
NC, NS = 2, 16
NW = NC * NS
ROWS_PER_W = NB_ // NW
RC = 2
NSTEP = ROWS_PER_W // RC


def _emb_body(text_hbm, table_hbm, out_hbm, idx_v, rows0, rows1,
              gs0, gs1, ws0, ws1):
    rows = (rows0, rows1)
    gsem = (gs0, gs1)
    wsem = (ws0, ws1)
    wid = lax.axis_index("s") * NC + lax.axis_index("c")
    row0 = wid * ROWS_PER_W

    pltpu.sync_copy(text_hbm.at[pl.ds(row0, ROWS_PER_W)], idx_v)

    def gather_start(s, b):
        for q in range(RC):
            pltpu.async_copy(
                table_hbm.at[idx_v.at[s * RC + q]], rows[b].at[q], gsem[b])

    def gather_wait(s, b):
        for q in range(RC):
            pltpu.make_async_copy(
                table_hbm.at[idx_v.at[s * RC + q]], rows[b].at[q],
                gsem[b]).wait()

    def write_start(s, b):
        pltpu.async_copy(
            rows[b], out_hbm.at[pl.ds(row0 + s * RC, RC)], wsem[b])

    def write_wait(s, b):
        pltpu.make_async_copy(
            rows[b], out_hbm.at[pl.ds(row0 + s * RC, RC)], wsem[b]).wait()

    gather_start(0, 0)
    gather_start(1, 1)

    def step(p, carry):
        for b in (0, 1):
            s = 2 * p + b
            gather_wait(s, b)
            write_start(s, b)
            write_wait(s, b)
            gather_start(s + 2, b)
        return carry

    lax.fori_loop(0, NSTEP // 2 - 1, step, 0)

    for b in (0, 1):
        s = NSTEP - 2 + b
        gather_wait(s, b)
        write_start(s, b)
    for b in (0, 1):
        write_wait(NSTEP - 2 + b, b)


@jax.jit
def _embed(text, table):
    mesh = plsc.VectorSubcoreMesh(core_axis_name="c", subcore_axis_name="s")
    f = functools.partial(
        pl.kernel,
        out_type=jax.ShapeDtypeStruct((NB_, NT, D), jnp.float32),
        mesh=mesh,
        scratch_types=[
            pltpu.VMEM((ROWS_PER_W, NT), jnp.int32),
            pltpu.VMEM((RC, NT, D), jnp.float32),
            pltpu.VMEM((RC, NT, D), jnp.float32),
            pltpu.SemaphoreType.DMA,
            pltpu.SemaphoreType.DMA,
            pltpu.SemaphoreType.DMA,
            pltpu.SemaphoreType.DMA,
        ],
        compiler_params=pltpu.CompilerParams(use_tc_tiling_on_sc=False),
    )(_emb_body)
    return f(text, table)


def kernel(text, seq_len, text_embed_weight):
    return _embed(text, text_embed_weight)


# speedup vs baseline: 1.2275x; 1.2275x over previous
import functools

import jax
import jax.numpy as jnp
from jax import lax
from jax.experimental import pallas as pl
from jax.experimental.pallas import tpu as pltpu
from jax.experimental.pallas import tpu_sc as plsc

NB_, NT = 4096, 200
D = 64
DP = 128
NC, NS = 2, 16
NW = NC * NS
ROWS_PER_W = NB_ // NW
RC = 1
NSTEP = ROWS_PER_W // RC


def _emb_body(text_hbm, table_hbm, out_hbm, idx_v, rows0, rows1,
              gs0, gs1, ws0, ws1):
    rows = (rows0, rows1)
    gsem = (gs0, gs1)
    wsem = (ws0, ws1)
    wid = lax.axis_index("s") * NC + lax.axis_index("c")
    row0 = wid * ROWS_PER_W

    pltpu.sync_copy(text_hbm.at[pl.ds(row0, ROWS_PER_W)], idx_v)

    def gather_start(s, b):
        for q in range(RC):
            pltpu.async_copy(
                table_hbm.at[idx_v.at[s * RC + q]], rows[b].at[q], gsem[b])

    def gather_wait(s, b):
        for q in range(RC):
            pltpu.make_async_copy(
                table_hbm.at[idx_v.at[s * RC + q]], rows[b].at[q],
                gsem[b]).wait()

    def write_start(s, b):
        pltpu.async_copy(
            rows[b], out_hbm.at[pl.ds(row0 + s * RC, RC)], wsem[b])

    def write_wait(s, b):
        pltpu.make_async_copy(
            rows[b], out_hbm.at[pl.ds(row0 + s * RC, RC)], wsem[b]).wait()

    gather_start(0, 0)
    gather_start(1, 1)

    def step(p, carry):
        for b in (0, 1):
            s = 2 * p + b
            gather_wait(s, b)
            write_start(s, b)
            write_wait(s, b)
            gather_start(s + 2, b)
        return carry

    lax.fori_loop(0, NSTEP // 2 - 1, step, 0)

    for b in (0, 1):
        s = NSTEP - 2 + b
        gather_wait(s, b)
        write_start(s, b)
    for b in (0, 1):
        write_wait(NSTEP - 2 + b, b)


@jax.jit
def _embed(text, table):
    table128 = jnp.pad(table, ((0, 0), (0, DP - D)))
    mesh = plsc.VectorSubcoreMesh(core_axis_name="c", subcore_axis_name="s")
    f = functools.partial(
        pl.kernel,
        out_type=jax.ShapeDtypeStruct((NB_, NT, DP), jnp.float32),
        mesh=mesh,
        scratch_types=[
            pltpu.VMEM((ROWS_PER_W, NT), jnp.int32),
            pltpu.VMEM((RC, NT, DP), jnp.float32),
            pltpu.VMEM((RC, NT, DP), jnp.float32),
            pltpu.SemaphoreType.DMA,
            pltpu.SemaphoreType.DMA,
            pltpu.SemaphoreType.DMA,
            pltpu.SemaphoreType.DMA,
        ],
        compiler_params=pltpu.CompilerParams(use_tc_tiling_on_sc=False),
    )(_emb_body)
    out128 = f(text, table128)
    return out128[:, :, :D]


def kernel(text, seq_len, text_embed_weight):
    return _embed(text, text_embed_weight)


# speedup vs baseline: 1.4380x; 1.1715x over previous
import functools

import jax
import jax.numpy as jnp
from jax import lax
from jax.experimental import pallas as pl
from jax.experimental.pallas import tpu as pltpu
from jax.experimental.pallas import tpu_sc as plsc

NB_, NT = 4096, 200
D = 64
DP = 128
VP = 1000008
NC, NS = 2, 16
NW = NC * NS
ROWS_PER_W = NB_ // NW
RC = 2
NSTEP = ROWS_PER_W // RC


def _emb_body(text2_hbm, table2_hbm, out_hbm, idx_v, rows0, rows1,
              gs0, gs1, ws0, ws1):
    rows = (rows0, rows1)
    gsem = (gs0, gs1)
    wsem = (ws0, ws1)
    wid = lax.axis_index("s") * NC + lax.axis_index("c")
    row0 = wid * ROWS_PER_W

    pltpu.sync_copy(text2_hbm.at[pl.ds(row0, ROWS_PER_W)], idx_v)

    def gather_start(s, b):
        for q in range(RC):
            pltpu.async_copy(
                table2_hbm.at[idx_v.at[s * RC + q]], rows[b].at[q], gsem[b])

    def gather_wait(s, b):
        for q in range(RC):
            pltpu.make_async_copy(
                table2_hbm.at[idx_v.at[s * RC + q]], rows[b].at[q],
                gsem[b]).wait()

    def write_start(s, b):
        pltpu.async_copy(
            rows[b], out_hbm.at[pl.ds(row0 + s * RC, RC), :, pl.ds(0, D)],
            wsem[b])

    def write_wait(s, b):
        pltpu.make_async_copy(
            rows[b], out_hbm.at[pl.ds(row0 + s * RC, RC), :, pl.ds(0, D)],
            wsem[b]).wait()

    gather_start(0, 0)
    gather_start(1, 1)

    def step(p, carry):
        for b in (0, 1):
            s = 2 * p + b
            gather_wait(s, b)
            write_start(s, b)
            write_wait(s, b)
            gather_start(s + 2, b)
        return carry

    lax.fori_loop(0, NSTEP // 2 - 1, step, 0)

    for b in (0, 1):
        s = NSTEP - 2 + b
        gather_wait(s, b)
        write_start(s, b)
    for b in (0, 1):
        write_wait(NSTEP - 2 + b, b)


@jax.jit
def _embed(text, table):
    table128 = jnp.pad(table, ((0, VP - table.shape[0]), (0, DP - D)))
    table2 = table128.reshape(2 * VP, D)
    text2 = text * 2
    mesh = plsc.VectorSubcoreMesh(core_axis_name="c", subcore_axis_name="s")
    f = functools.partial(
        pl.kernel,
        out_type=jax.ShapeDtypeStruct((NB_, NT, DP), jnp.float32),
        mesh=mesh,
        scratch_types=[
            pltpu.VMEM((ROWS_PER_W, NT), jnp.int32),
            pltpu.VMEM((RC, NT, D), jnp.float32),
            pltpu.VMEM((RC, NT, D), jnp.float32),
            pltpu.SemaphoreType.DMA,
            pltpu.SemaphoreType.DMA,
            pltpu.SemaphoreType.DMA,
            pltpu.SemaphoreType.DMA,
        ],
        compiler_params=pltpu.CompilerParams(use_tc_tiling_on_sc=False),
    )(_emb_body)
    out128 = f(text2, table2)
    return out128[:, :, :D]


def kernel(text, seq_len, text_embed_weight):
    return _embed(text, text_embed_weight)


# speedup vs baseline: 1.4426x; 1.0032x over previous
import functools

import jax
import jax.numpy as jnp
from jax import lax
from jax.experimental import pallas as pl
from jax.experimental.pallas import tpu as pltpu
from jax.experimental.pallas import tpu_sc as plsc

NB_, NT = 4096, 200
D = 64
DP = 128
V = 1000001
VP = 1000008
NC, NS = 2, 16
NW = NC * NS
ROWS_PER_W = NB_ // NW
RC = 2
NSTEP = ROWS_PER_W // RC

BV = 2048
PGRID = -(-VP // BV)


def _tp_body(t_ref, out_ref):
    tt = jnp.swapaxes(t_ref[...], 0, 1)
    out_ref[:, :D] = tt
    out_ref[:, D:] = tt


def _emb_body(text2_hbm, table2_hbm, out_hbm, idx_v, rows0, rows1,
              gs0, gs1, ws0, ws1):
    rows = (rows0, rows1)
    gsem = (gs0, gs1)
    wsem = (ws0, ws1)
    wid = lax.axis_index("s") * NC + lax.axis_index("c")
    row0 = wid * ROWS_PER_W

    pltpu.sync_copy(text2_hbm.at[pl.ds(row0, ROWS_PER_W)], idx_v)

    def gather_start(s, b):
        for q in range(RC):
            pltpu.async_copy(
                table2_hbm.at[idx_v.at[s * RC + q]], rows[b].at[q], gsem[b])

    def gather_wait(s, b):
        for q in range(RC):
            pltpu.make_async_copy(
                table2_hbm.at[idx_v.at[s * RC + q]], rows[b].at[q],
                gsem[b]).wait()

    def write_start(s, b):
        pltpu.async_copy(
            rows[b], out_hbm.at[pl.ds(row0 + s * RC, RC), :, pl.ds(0, D)],
            wsem[b])

    def write_wait(s, b):
        pltpu.make_async_copy(
            rows[b], out_hbm.at[pl.ds(row0 + s * RC, RC), :, pl.ds(0, D)],
            wsem[b]).wait()

    gather_start(0, 0)
    gather_start(1, 1)

    def step(p, carry):
        for b in (0, 1):
            s = 2 * p + b
            gather_wait(s, b)
            write_start(s, b)
            write_wait(s, b)
            gather_start(s + 2, b)
        return carry

    lax.fori_loop(0, NSTEP // 2 - 1, step, 0)

    for b in (0, 1):
        s = NSTEP - 2 + b
        gather_wait(s, b)
        write_start(s, b)
    for b in (0, 1):
        write_wait(NSTEP - 2 + b, b)


@jax.jit
def _embed(text, table):
    mesh = plsc.VectorSubcoreMesh(core_axis_name="c", subcore_axis_name="s")
    table128 = pl.pallas_call(
        _tp_body,
        grid=(PGRID,),
        in_specs=[pl.BlockSpec((D, BV), lambda i: (0, i))],
        out_specs=pl.BlockSpec((BV, DP), lambda i: (i, 0)),
        out_shape=jax.ShapeDtypeStruct((VP, DP), jnp.float32),
    )(jnp.swapaxes(table, 0, 1))
    table2 = table128.reshape(2 * VP, D)
    text2 = text * 2
    f = functools.partial(
        pl.kernel,
        out_type=jax.ShapeDtypeStruct((NB_, NT, DP), jnp.float32),
        mesh=mesh,
        scratch_types=[
            pltpu.VMEM((ROWS_PER_W, NT), jnp.int32),
            pltpu.VMEM((RC, NT, D), jnp.float32),
            pltpu.VMEM((RC, NT, D), jnp.float32),
            pltpu.SemaphoreType.DMA,
            pltpu.SemaphoreType.DMA,
            pltpu.SemaphoreType.DMA,
            pltpu.SemaphoreType.DMA,
        ],
        compiler_params=pltpu.CompilerParams(use_tc_tiling_on_sc=False),
    )(_emb_body)
    out128 = f(text2, table2)
    return out128[:, :, :D]


def kernel(text, seq_len, text_embed_weight):
    return _embed(text, text_embed_weight)


# speedup vs baseline: 1.4429x; 1.0002x over previous
import functools

import jax
import jax.numpy as jnp
from jax import lax
from jax.experimental import pallas as pl
from jax.experimental.pallas import tpu as pltpu
from jax.experimental.pallas import tpu_sc as plsc

NB_, NT = 4096, 200
D = 64
DP = 128
VP = 1000008
NC, NS = 2, 16
NW = NC * NS
ROWS_PER_W = NB_ // NW
RC = 2
NSTEP = ROWS_PER_W // RC

BV = 2048
PGRID = -(-VP // BV)


def _tp_body(t_ref, out_ref):
    tt = jnp.swapaxes(t_ref[...], 0, 1)
    out_ref[:, :D] = tt
    out_ref[:, D:] = tt


def _emb_body(text2_hbm, table2_hbm, out_hbm, idx_v, rows0, rows1,
              gs0, gs1, ws0, ws1):
    rows = (rows0, rows1)
    gsem = (gs0, gs1)
    wsem = (ws0, ws1)
    wid = lax.axis_index("s") * NC + lax.axis_index("c")
    row0 = wid * ROWS_PER_W

    pltpu.sync_copy(text2_hbm.at[pl.ds(row0, ROWS_PER_W)], idx_v)

    def gather_start(s, b):
        for q in range(RC):
            pltpu.async_copy(
                table2_hbm.at[idx_v.at[s * RC + q]], rows[b].at[q], gsem[b])

    def gather_wait(s, b):
        for q in range(RC):
            pltpu.make_async_copy(
                table2_hbm.at[idx_v.at[s * RC + q]], rows[b].at[q],
                gsem[b]).wait()

    def write_start(s, b):
        pltpu.async_copy(
            rows[b], out_hbm.at[pl.ds(row0 + s * RC, RC), :, pl.ds(0, D)],
            wsem[b])

    def write_wait(s, b):
        pltpu.make_async_copy(
            rows[b], out_hbm.at[pl.ds(row0 + s * RC, RC), :, pl.ds(0, D)],
            wsem[b]).wait()

    gather_start(0, 0)
    gather_start(1, 1)

    def step(p, carry):
        for b in (0, 1):
            s = 2 * p + b
            gather_wait(s, b)
            write_start(s, b)
            write_wait(s, b)
            gather_start(s + 2, b)
        return carry

    lax.fori_loop(0, NSTEP // 2 - 1, step, 0)

    for b in (0, 1):
        s = NSTEP - 2 + b
        gather_wait(s, b)
        write_start(s, b)
    for b in (0, 1):
        write_wait(NSTEP - 2 + b, b)


@jax.jit
def _embed(text, table):
    mesh = plsc.VectorSubcoreMesh(core_axis_name="c", subcore_axis_name="s")
    table128 = pl.pallas_call(
        _tp_body,
        grid=(PGRID,),
        in_specs=[pl.BlockSpec((D, BV), lambda i: (0, i))],
        out_specs=pl.BlockSpec((BV, DP), lambda i: (i, 0)),
        out_shape=jax.ShapeDtypeStruct((VP, DP), jnp.float32),
    )(jnp.swapaxes(table, 0, 1))
    table2 = table128.reshape(2 * VP, D)
    text2 = text * 2
    f = functools.partial(
        pl.kernel,
        out_type=jax.ShapeDtypeStruct((NB_, NT, DP), jnp.float32),
        mesh=mesh,
        scratch_types=[
            pltpu.VMEM((ROWS_PER_W, NT), jnp.int32),
            pltpu.VMEM((RC, NT, D), jnp.float32),
            pltpu.VMEM((RC, NT, D), jnp.float32),
            pltpu.SemaphoreType.DMA,
            pltpu.SemaphoreType.DMA,
            pltpu.SemaphoreType.DMA,
            pltpu.SemaphoreType.DMA,
        ],
        compiler_params=pltpu.CompilerParams(use_tc_tiling_on_sc=False),
    )(_emb_body)
    out128 = f(text2, table2)
    return out128[:, :, :D]


def kernel(text, seq_len, text_embed_weight):
    return _embed(text, text_embed_weight)


# speedup vs baseline: 1.8186x; 1.2604x over previous
import functools

import jax
import jax.numpy as jnp
from jax import lax
from jax.experimental import pallas as pl
from jax.experimental.pallas import tpu as pltpu
from jax.experimental.pallas import tpu_sc as plsc

NB_, NT = 4096, 200
D = 64
DP = 128
VP = 1000008
NC, NS = 2, 16
NW = NC * NS
ROWS_PER_W = NB_ // NW
RC = 2
NSTEP = ROWS_PER_W // RC

BV = 4096
PGRID = -(-VP // BV)


def _tp_body(t_ref, out_ref):
    tt = jnp.swapaxes(t_ref[...], 0, 1)
    out_ref[:, :D] = tt


def _emb_body(text2_hbm, table2_hbm, out_hbm, idx_v, rows0, rows1,
              gs0, gs1, ws0, ws1):
    rows = (rows0, rows1)
    gsem = (gs0, gs1)
    wsem = (ws0, ws1)
    wid = lax.axis_index("s") * NC + lax.axis_index("c")
    row0 = wid * ROWS_PER_W

    pltpu.sync_copy(text2_hbm.at[pl.ds(row0, ROWS_PER_W)], idx_v)

    def gather_start(s, b):
        for q in range(RC):
            pltpu.async_copy(
                table2_hbm.at[idx_v.at[s * RC + q]], rows[b].at[q], gsem[b])

    def gather_wait(s, b):
        for q in range(RC):
            pltpu.make_async_copy(
                table2_hbm.at[idx_v.at[s * RC + q]], rows[b].at[q],
                gsem[b]).wait()

    def write_start(s, b):
        pltpu.async_copy(
            rows[b], out_hbm.at[pl.ds(row0 + s * RC, RC), :, pl.ds(0, D)],
            wsem[b])

    def write_wait(s, b):
        pltpu.make_async_copy(
            rows[b], out_hbm.at[pl.ds(row0 + s * RC, RC), :, pl.ds(0, D)],
            wsem[b]).wait()

    gather_start(0, 0)
    gather_start(1, 1)

    def step(p, carry):
        for b in (0, 1):
            s = 2 * p + b
            gather_wait(s, b)
            write_start(s, b)
            write_wait(s, b)
            gather_start(s + 2, b)
        return carry

    lax.fori_loop(0, NSTEP // 2 - 1, step, 0)

    for b in (0, 1):
        s = NSTEP - 2 + b
        gather_wait(s, b)
        write_start(s, b)
    for b in (0, 1):
        write_wait(NSTEP - 2 + b, b)


@jax.jit
def _embed(text, table):
    mesh = plsc.VectorSubcoreMesh(core_axis_name="c", subcore_axis_name="s")
    table128 = pl.pallas_call(
        _tp_body,
        grid=(PGRID,),
        in_specs=[pl.BlockSpec((D, BV), lambda i: (0, i))],
        out_specs=pl.BlockSpec((BV, DP), lambda i: (i, 0)),
        out_shape=jax.ShapeDtypeStruct((VP, DP), jnp.float32),
    )(jnp.swapaxes(table, 0, 1))
    table2 = table128.reshape(2 * VP, D)
    text2 = text * 2
    f = functools.partial(
        pl.kernel,
        out_type=jax.ShapeDtypeStruct((NB_, NT, DP), jnp.float32),
        mesh=mesh,
        scratch_types=[
            pltpu.VMEM((ROWS_PER_W, NT), jnp.int32),
            pltpu.VMEM((RC, NT, D), jnp.float32),
            pltpu.VMEM((RC, NT, D), jnp.float32),
            pltpu.SemaphoreType.DMA,
            pltpu.SemaphoreType.DMA,
            pltpu.SemaphoreType.DMA,
            pltpu.SemaphoreType.DMA,
        ],
        compiler_params=pltpu.CompilerParams(use_tc_tiling_on_sc=False),
    )(_emb_body)
    out128 = f(text2, table2)
    return out128[:, :, :D]


def kernel(text, seq_len, text_embed_weight):
    return _embed(text, text_embed_weight)


# speedup vs baseline: 2.0381x; 1.1207x over previous
import functools

import jax
import jax.numpy as jnp
from jax import lax
from jax.experimental import pallas as pl
from jax.experimental.pallas import tpu as pltpu
from jax.experimental.pallas import tpu_sc as plsc

NB_, NT = 4096, 200
D = 64
DP = 128
VP = 1000008
NC, NS = 2, 16
NW = NC * NS
ROWS_PER_W = NB_ // NW
RC = 2
NSTEP = ROWS_PER_W // RC

BV = 8192
PGRID = -(-VP // BV)


def _tp_body(t_ref, out_ref):
    out_ref[:, :D] = jnp.swapaxes(t_ref[...], 0, 1)


def _emb_body(text2_hbm, table2_hbm, out_hbm, idx_v, rows0, rows1,
              gs0, gs1, ws0, ws1):
    rows = (rows0, rows1)
    gsem = (gs0, gs1)
    wsem = (ws0, ws1)
    wid = lax.axis_index("s") * NC + lax.axis_index("c")
    row0 = wid * ROWS_PER_W

    pltpu.sync_copy(text2_hbm.at[pl.ds(row0, ROWS_PER_W)], idx_v)

    def gather_start(s, b):
        for q in range(RC):
            pltpu.async_copy(
                table2_hbm.at[idx_v.at[s * RC + q]], rows[b].at[q], gsem[b])

    def gather_wait(s, b):
        for q in range(RC):
            pltpu.make_async_copy(
                table2_hbm.at[idx_v.at[s * RC + q]], rows[b].at[q],
                gsem[b]).wait()

    def write_start(s, b):
        pltpu.async_copy(
            rows[b], out_hbm.at[pl.ds(row0 + s * RC, RC), :, pl.ds(0, D)],
            wsem[b])

    def write_wait(s, b):
        pltpu.make_async_copy(
            rows[b], out_hbm.at[pl.ds(row0 + s * RC, RC), :, pl.ds(0, D)],
            wsem[b]).wait()

    gather_start(0, 0)
    gather_start(1, 1)

    def step(p, carry):
        for b in (0, 1):
            s = 2 * p + b
            gather_wait(s, b)
            write_start(s, b)
            write_wait(s, b)
            gather_start(s + 2, b)
        return carry

    lax.fori_loop(0, NSTEP // 2 - 1, step, 0)

    for b in (0, 1):
        s = NSTEP - 2 + b
        gather_wait(s, b)
        write_start(s, b)
    for b in (0, 1):
        write_wait(NSTEP - 2 + b, b)


@jax.jit
def _embed(text, table):
    mesh = plsc.VectorSubcoreMesh(core_axis_name="c", subcore_axis_name="s")
    table128 = pl.pallas_call(
        _tp_body,
        grid=(PGRID,),
        in_specs=[pl.BlockSpec((D, BV), lambda i: (0, i))],
        out_specs=pl.BlockSpec((BV, DP), lambda i: (i, 0)),
        out_shape=jax.ShapeDtypeStruct((VP, DP), jnp.float32),
    )(jnp.swapaxes(table, 0, 1))
    table2 = table128.reshape(2 * VP, D)
    text2 = text * 2
    f = functools.partial(
        pl.kernel,
        out_type=jax.ShapeDtypeStruct((NB_, NT, DP), jnp.float32),
        mesh=mesh,
        scratch_types=[
            pltpu.VMEM((ROWS_PER_W, NT), jnp.int32),
            pltpu.VMEM((RC, NT, D), jnp.float32),
            pltpu.VMEM((RC, NT, D), jnp.float32),
            pltpu.SemaphoreType.DMA,
            pltpu.SemaphoreType.DMA,
            pltpu.SemaphoreType.DMA,
            pltpu.SemaphoreType.DMA,
        ],
        compiler_params=pltpu.CompilerParams(use_tc_tiling_on_sc=False),
    )(_emb_body)
    out128 = f(text2, table2)
    return out128[:, :, :D]


def kernel(text, seq_len, text_embed_weight):
    return _embed(text, text_embed_weight)


# speedup vs baseline: 2.1028x; 1.0317x over previous
import functools

import jax
import jax.numpy as jnp
from jax import lax
from jax.experimental import pallas as pl
from jax.experimental.pallas import tpu as pltpu
from jax.experimental.pallas import tpu_sc as plsc

NB_, NT = 4096, 200
D = 64
DP = 128
VP = 1000008
NC, NS = 2, 16
NW = NC * NS
ROWS_PER_W = NB_ // NW
RC = 2
NSTEP = ROWS_PER_W // RC

BV = 16384
PGRID = -(-VP // BV)


def _tp_body(t_ref, out_ref):
    out_ref[:, :D] = jnp.swapaxes(t_ref[...], 0, 1)


def _emb_body(text2_hbm, table2_hbm, out_hbm, idx_v, rows0, rows1,
              gs0, gs1, ws0, ws1):
    rows = (rows0, rows1)
    gsem = (gs0, gs1)
    wsem = (ws0, ws1)
    wid = lax.axis_index("s") * NC + lax.axis_index("c")
    row0 = wid * ROWS_PER_W

    pltpu.sync_copy(text2_hbm.at[pl.ds(row0, ROWS_PER_W)], idx_v)

    def gather_start(s, b):
        for q in range(RC):
            pltpu.async_copy(
                table2_hbm.at[idx_v.at[s * RC + q]], rows[b].at[q], gsem[b])

    def gather_wait(s, b):
        for q in range(RC):
            pltpu.make_async_copy(
                table2_hbm.at[idx_v.at[s * RC + q]], rows[b].at[q],
                gsem[b]).wait()

    def write_start(s, b):
        pltpu.async_copy(
            rows[b], out_hbm.at[pl.ds(row0 + s * RC, RC), :, pl.ds(0, D)],
            wsem[b])

    def write_wait(s, b):
        pltpu.make_async_copy(
            rows[b], out_hbm.at[pl.ds(row0 + s * RC, RC), :, pl.ds(0, D)],
            wsem[b]).wait()

    gather_start(0, 0)
    gather_start(1, 1)

    def step(p, carry):
        for b in (0, 1):
            s = 2 * p + b
            gather_wait(s, b)
            write_start(s, b)
            write_wait(s, b)
            gather_start(s + 2, b)
        return carry

    lax.fori_loop(0, NSTEP // 2 - 1, step, 0)

    for b in (0, 1):
        s = NSTEP - 2 + b
        gather_wait(s, b)
        write_start(s, b)
    for b in (0, 1):
        write_wait(NSTEP - 2 + b, b)


@jax.jit
def _embed(text, table):
    mesh = plsc.VectorSubcoreMesh(core_axis_name="c", subcore_axis_name="s")
    table128 = pl.pallas_call(
        _tp_body,
        grid=(PGRID,),
        in_specs=[pl.BlockSpec((D, BV), lambda i: (0, i))],
        out_specs=pl.BlockSpec((BV, DP), lambda i: (i, 0)),
        out_shape=jax.ShapeDtypeStruct((VP, DP), jnp.float32),
    )(jnp.swapaxes(table, 0, 1))
    table2 = table128.reshape(2 * VP, D)
    text2 = text * 2
    f = functools.partial(
        pl.kernel,
        out_type=jax.ShapeDtypeStruct((NB_, NT, DP), jnp.float32),
        mesh=mesh,
        scratch_types=[
            pltpu.VMEM((ROWS_PER_W, NT), jnp.int32),
            pltpu.VMEM((RC, NT, D), jnp.float32),
            pltpu.VMEM((RC, NT, D), jnp.float32),
            pltpu.SemaphoreType.DMA,
            pltpu.SemaphoreType.DMA,
            pltpu.SemaphoreType.DMA,
            pltpu.SemaphoreType.DMA,
        ],
        compiler_params=pltpu.CompilerParams(use_tc_tiling_on_sc=False),
    )(_emb_body)
    out128 = f(text2, table2)
    return out128[:, :, :D]


def kernel(text, seq_len, text_embed_weight):
    return _embed(text, text_embed_weight)
